# baseline (speedup 1.0000x reference)
import functools

import jax
import jax.numpy as jnp
import numpy as np
from jax import lax
from jax.experimental import pallas as pl
from jax.experimental.pallas import tpu as pltpu
from jax.experimental.pallas import tpu_sc as plsc

N = 20000
G = 128
N_FG = 5000
N_BG = 15000
BN = 2048

_K05 = int(~np.float32(0.5).view(np.int32))
_K01 = int(~np.float32(0.1).view(np.int32))
_K1 = int(~np.float32(1.0).view(np.int32))
_RAD0 = 1 << 13
_RAD1 = 1 << 12

_D0S = (_K01 - _K1) & (_RAD0 - 1)
_D0S_ROW = (_D0S // 16) * 16
_D0S_LANE = _D0S % 16

_NV = N // 16
_NV2 = _NV // 2
_CH = 4000
_NCH = N // _CH


def _tc_body(roist_ref, gtall_ref, dx_ref, dy_ref, dw_ref, dh_ref,
             key_ref, lab_ref):
    r = roist_ref[...]
    x1, y1, x2, y2 = (r[i : i + 1, :] for i in range(4))
    g = gtall_ref[...]
    gx1, gy1, gx2, gy2 = (g[:, i : i + 1] for i in range(4))
    ix1 = jnp.maximum(x1, gx1)
    iy1 = jnp.maximum(y1, gy1)
    ix2 = jnp.minimum(x2, gx2)
    iy2 = jnp.minimum(y2, gy2)
    inter = jnp.maximum(ix2 - ix1, 0.0) * jnp.maximum(iy2 - iy1, 0.0)
    area_b = (x2 - x1) * (y2 - y1)
    area_g = (gx2 - gx1) * (gy2 - gy1)
    union = area_b + area_g - inter
    iou = inter / jnp.maximum(union, 1e-8)
    mx = jnp.max(iou, axis=0, keepdims=True)
    iota_g = lax.broadcasted_iota(jnp.int32, iou.shape, 0).astype(jnp.float32)
    am = jnp.min(jnp.where(iou == mx, iota_g, jnp.float32(G)), axis=0,
                 keepdims=True)
    onehot = iota_g == am

    def sel(c):
        col = g[:, c : c + 1]
        return jnp.sum(jnp.where(onehot, col, 0.0), axis=0, keepdims=True)

    mgx1, mgy1, mgx2, mgy2, labf = (sel(c) for c in range(5))
    ew = x2 - x1
    eh = y2 - y1
    ecx = x1 + 0.5 * ew
    ecy = y1 + 0.5 * eh
    gw = mgx2 - mgx1
    gh = mgy2 - mgy1
    gcx = mgx1 + 0.5 * gw
    gcy = mgy1 + 0.5 * gh
    ewc = jnp.maximum(ew, 1e-6)
    ehc = jnp.maximum(eh, 1e-6)
    dx_ref[...] = (gcx - ecx) / ewc / 0.1
    dy_ref[...] = (gcy - ecy) / ehc / 0.1
    dw_ref[...] = jnp.log(jnp.maximum(gw, 1e-6) / ewc) / 0.2
    dh_ref[...] = jnp.log(jnp.maximum(gh, 1e-6) / ehc) / 0.2
    key_ref[...] = ~lax.bitcast_convert_type(mx, jnp.int32)
    lab_ref[...] = labf.astype(jnp.int32)


def _tc_stage(roist, gtall):
    return pl.pallas_call(
        _tc_body,
        grid=((N + BN - 1) // BN,),
        in_specs=[
            pl.BlockSpec((4, BN), lambda i: (0, i)),
            pl.BlockSpec((G, 8), lambda i: (0, 0)),
        ],
        out_specs=[pl.BlockSpec((1, BN), lambda i: (0, i))] * 6,
        out_shape=[jax.ShapeDtypeStruct((1, N), jnp.float32)] * 4
        + [
            jax.ShapeDtypeStruct((1, N), jnp.int32),
            jax.ShapeDtypeStruct((1, N), jnp.int32),
        ],
    )(roist, gtall)


def _digit(k, p):
    kb = k - jnp.int32(_K1)
    if p == 0:
        return kb & jnp.int32(_RAD0 - 1)
    return lax.shift_right_logical(kb, jnp.int32(13))


def _scan_bins(hist, base, nbins):
    def scan_body(h, run):
        v0 = hist[pl.ds(h * 32, 16)]
        v1 = hist[pl.ds(h * 32 + 16, 16)]
        c0 = plsc.cumsum(v0)
        c1 = plsc.cumsum(v1)
        s0 = c0[15]
        base[pl.ds(h * 32, 16)] = run + c0 - v0
        base[pl.ds(h * 32 + 16, 16)] = (run + s0) + c1 - v1
        return run + s0 + c1[15]

    lax.fori_loop(0, nbins // 32, scan_body, jnp.int32(0))


def _perm_pass(src_k, src_v, dst_k, dst_v, base, p, ntv2, hist1=None):
    def perm_half(k, v):
        d = _digit(k, p)
        occ, last = plsc.scan_count(d)
        b = plsc.load_gather(base, [d])
        pos = b + occ - 1
        plsc.store_scatter(dst_k, [pos], k)
        plsc.store_scatter(dst_v, [pos], v)
        plsc.addupdate_scatter(base, [d], occ, mask=last)
        if hist1 is not None:
            d1 = _digit(k, 1)
            occ1, last1 = plsc.scan_count(d1)
            plsc.addupdate_scatter(hist1, [d1], occ1, mask=last1)

    def perm_body(t, _):
        k0 = src_k[pl.ds(t * 32, 16)]
        v0 = src_v[pl.ds(t * 32, 16)]
        k1 = src_k[pl.ds(t * 32 + 16, 16)]
        v1 = src_v[pl.ds(t * 32 + 16, 16)]
        perm_half(k0, v0)
        perm_half(k1, v1)
        return 0

    lax.fori_loop(0, ntv2, perm_body, 0)


def _sc_stage(key, zeros, dxa, dya, dwa, dha, labels_all):
    mesh = plsc.VectorSubcoreMesh(core_axis_name="c", subcore_axis_name="s")

    @functools.partial(
        pl.kernel,
        mesh=mesh,
        compiler_params=pltpu.CompilerParams(needs_layout_passes=False),
        out_type=(
            jax.ShapeDtypeStruct((N,), jnp.float32),
            jax.ShapeDtypeStruct((N,), jnp.float32),
            jax.ShapeDtypeStruct((N,), jnp.float32),
            jax.ShapeDtypeStruct((N,), jnp.float32),
            jax.ShapeDtypeStruct((N,), jnp.int32),
        ),
        scratch_types=[
            pltpu.VMEM((N + 64,), jnp.int32),
            pltpu.VMEM((N + 64,), jnp.int32),
            pltpu.VMEM((N + 64,), jnp.int32),
            pltpu.VMEM((N + 64,), jnp.int32),
            pltpu.VMEM((_RAD0,), jnp.int32),
            pltpu.VMEM((_RAD0,), jnp.int32),
            pltpu.VMEM((_RAD1,), jnp.int32),
            pltpu.VMEM((_CH,), jnp.float32),
            pltpu.VMEM((_CH,), jnp.float32),
            pltpu.VMEM((_CH,), jnp.float32),
            pltpu.VMEM((_CH,), jnp.float32),
            pltpu.VMEM((_CH,), jnp.int32),
            pltpu.VMEM((_CH,), jnp.int32),
            pltpu.SemaphoreType.DMA,
            pltpu.SemaphoreType.DMA,
            pltpu.SemaphoreType.DMA,
            pltpu.SemaphoreType.DMA,
            pltpu.SemaphoreType.DMA,
        ],
    )
    def sck(key_hbm, zeros_hbm, dx_hbm, dy_hbm, dw_hbm, dh_hbm, lall_hbm,
            dx_out, dy_out, dw_out, dh_out, lout_hbm,
            kB, vB, kC, vC, hist, base, hist1, c0, c1, c2, c3, lrows, lzero,
            sem0, sem1, sem2, sem3, sem4):
        cid = lax.axis_index("c")
        sid = lax.axis_index("s")

        @pl.when(jnp.logical_and(cid == 0, sid == 0))
        def _():
            iota16 = lax.iota(jnp.int32, 16)
            pltpu.sync_copy(zeros_hbm.at[pl.ds(0, _RAD0)], hist)
            pltpu.sync_copy(zeros_hbm.at[pl.ds(_RAD0, _RAD1)], hist1)
            pltpu.sync_copy(key_hbm, kC.at[pl.ds(0, N)])

            def compact_half(t16, off):
                k = kC[pl.ds(t16, 16)]
                sel = jnp.logical_and(
                    k < jnp.int32(_K01), k != jnp.int32(_K05)
                )
                iv = iota16 + t16
                plsc.store_compressed(kB.at[pl.ds(off, 16)], k, mask=sel)
                plsc.store_compressed(vB.at[pl.ds(off, 16)], iv, mask=sel)
                d0 = _digit(k, 0)
                occ, last = plsc.scan_count(d0, sel)
                plsc.addupdate_scatter(
                    hist, [d0], occ, mask=jnp.logical_and(last, sel)
                )
                return off + jnp.sum(sel.astype(jnp.int32))

            def compact_body(t, off):
                off = compact_half(t * 32, off)
                return compact_half(t * 32 + 16, off)

            M = lax.fori_loop(0, _NV2, compact_body, jnp.int32(0))
            mf = (M // 16) * 16
            tailv = kB[pl.ds(mf, 16)]
            kB[pl.ds(mf, 16)] = jnp.where(
                iota16 < M - mf, tailv, jnp.int32(_K01)
            )
            kB[pl.ds(mf + 16, 16)] = jnp.full((16,), _K01, jnp.int32)
            ntv2 = (M + 31) // 32
            nsent = ntv2 * 32 - M
            hv = hist[pl.ds(_D0S_ROW, 16)]
            hist[pl.ds(_D0S_ROW, 16)] = hv + jnp.where(
                iota16 == jnp.int32(_D0S_LANE), nsent, 0
            )

            _scan_bins(hist, base, _RAD0)
            _perm_pass(kB, vB, kC, vC, base, 0, ntv2, hist1=hist1)
            _scan_bins(hist1, hist, _RAD1)
            _perm_pass(kC, vC, kB, vB, hist, 1, ntv2)
            pltpu.sync_copy(key_hbm, kC.at[pl.ds(0, N)])

            idxs = vC

            def pick_half(t16, fr, br):
                k = kB[pl.ds(t16, 16)]
                v = vB[pl.ds(t16, 16)]
                is_fg = k < jnp.int32(_K05)
                is_bg = jnp.logical_and(
                    k > jnp.int32(_K05), k < jnp.int32(_K01)
                )
                cf = plsc.cumsum(is_fg.astype(jnp.int32))
                cb = plsc.cumsum(is_bg.astype(jnp.int32))
                rf = fr + cf - 1
                rb = br + cb - 1
                plsc.store_scatter(
                    idxs, [rf], v,
                    mask=jnp.logical_and(is_fg, rf < jnp.int32(N_FG)),
                )
                plsc.store_scatter(
                    idxs, [rb + jnp.int32(N_FG)], v,
                    mask=jnp.logical_and(is_bg, rb < jnp.int32(N_BG)),
                )
                return fr + cf[15], br + cb[15]

            def pick_body(t, fb):
                fr, br = pick_half(t * 32, fb[0], fb[1])
                return pick_half(t * 32 + 16, fr, br)

            F, B = lax.fori_loop(
                0, ntv2, pick_body, (jnp.int32(0), jnp.int32(0))
            )

            def fill_half(t16, nf, nb):
                k = kC[pl.ds(t16, 16)]
                iv = iota16 + t16
                no_fg = k >= jnp.int32(_K05)
                no_bg = jnp.logical_or(
                    k <= jnp.int32(_K05), k >= jnp.int32(_K01)
                )
                cf = plsc.cumsum(no_fg.astype(jnp.int32))
                cb = plsc.cumsum(no_bg.astype(jnp.int32))
                pf = F + nf + cf - 1
                pb = B + nb + cb - 1
                plsc.store_scatter(
                    idxs, [pf], iv,
                    mask=jnp.logical_and(no_fg, pf < jnp.int32(N_FG)),
                )
                plsc.store_scatter(
                    idxs, [pb + jnp.int32(N_FG)], iv,
                    mask=jnp.logical_and(no_bg, pb < jnp.int32(N_BG)),
                )
                return nf + cf[15], nb + cb[15]

            need_f = jnp.maximum(jnp.int32(N_FG) - F, 0)
            need_b = jnp.maximum(jnp.int32(N_BG) - B, 0)

            def fill_cond(c):
                t, nf, nb = c
                return jnp.logical_and(
                    t < _NV2,
                    jnp.logical_or(nf < need_f, nb < need_b),
                )

            def fill_body(c):
                t, nf, nb = c
                nf, nb = fill_half(t * 32, nf, nb)
                nf, nb = fill_half(t * 32 + 16, nf, nb)
                return (t + 1, nf, nb)

            lax.while_loop(
                fill_cond, fill_body,
                (jnp.int32(0), jnp.int32(0), jnp.int32(0)),
            )

            fthr = jnp.minimum(F, jnp.int32(N_FG))

            def zero_body(t, _):
                lzero[pl.ds(t * 16, 16)] = jnp.zeros((16,), jnp.int32)
                return 0

            lax.fori_loop(0, _CH // 16, zero_body, 0)

            def out_body(c, _):
                gi = idxs.at[pl.ds(c * _CH, _CH)]
                cps = [
                    pltpu.make_async_copy(src.at[gi], dst, sem)
                    for src, dst, sem in (
                        (dx_hbm, c0, sem0),
                        (dy_hbm, c1, sem1),
                        (dw_hbm, c2, sem2),
                        (dh_hbm, c3, sem3),
                    )
                ]
                for cp in cps:
                    cp.start()
                need_lab = c * _CH < fthr
                sl = pl.ds(c * _CH, _CH)

                @pl.when(need_lab)
                def _():
                    cp4 = pltpu.make_async_copy(
                        lall_hbm.at[gi], lrows, sem4
                    )
                    cp4.start()
                    cp4.wait()

                    def mask_body(t, _):
                        pvec = iota16 + (c * _CH + t * 16)
                        lv = lrows[pl.ds(t * 16, 16)]
                        lrows[pl.ds(t * 16, 16)] = jnp.where(
                            pvec < fthr, lv, jnp.int32(0)
                        )
                        return 0

                    lax.fori_loop(0, _CH // 16, mask_body, 0)
                    pltpu.sync_copy(lrows, lout_hbm.at[sl])

                @pl.when(jnp.logical_not(need_lab))
                def _():
                    pltpu.sync_copy(lzero, lout_hbm.at[sl])

                for cp in cps:
                    cp.wait()
                pltpu.sync_copy(c0, dx_out.at[sl])
                pltpu.sync_copy(c1, dy_out.at[sl])
                pltpu.sync_copy(c2, dw_out.at[sl])
                pltpu.sync_copy(c3, dh_out.at[sl])
                return 0

            lax.fori_loop(0, _NCH, out_body, 0)

    return sck(key, zeros, dxa, dya, dwa, dha, labels_all)


def kernel(rois, gt_bbox, gt_labels):
    gtall = jnp.concatenate(
        [gt_bbox, gt_labels.astype(jnp.float32)[:, None],
         jnp.zeros((G, 3), jnp.float32)],
        axis=1,
    )
    dxa, dya, dwa, dha, key2d, lab2d = _tc_stage(rois.T, gtall)
    key = key2d.reshape(N)
    labels_all = lab2d.reshape(N)
    zeros = jnp.zeros((_RAD0 + _RAD1,), jnp.int32)
    dx, dy, dw, dh, labels = _sc_stage(
        key, zeros, dxa.reshape(N), dya.reshape(N), dwa.reshape(N),
        dha.reshape(N), labels_all)
    deltas = jnp.stack([dx, dy, dw, dh], axis=1)
    return deltas, labels


# speedup vs baseline: 1.0107x; 1.0107x over previous
import functools

import jax
import jax.numpy as jnp
import numpy as np
from jax import lax
from jax.experimental import pallas as pl
from jax.experimental.pallas import tpu as pltpu
from jax.experimental.pallas import tpu_sc as plsc

N = 20000
G = 128
N_FG = 5000
N_BG = 15000
BN = 2048

_K05 = int(~np.float32(0.5).view(np.int32))
_K01 = int(~np.float32(0.1).view(np.int32))
_K1 = int(~np.float32(1.0).view(np.int32))
_RAD0 = 1 << 13
_RAD1 = 1 << 12

_D0S = (_K01 - _K1) & (_RAD0 - 1)
_D0S_ROW = (_D0S // 16) * 16
_D0S_LANE = _D0S % 16

_NV = N // 16
_NV2 = _NV // 2
_CH = 4000
_NCH = N // _CH


def _tc_body(roist_ref, gtall_ref, dx_ref, dy_ref, dw_ref, dh_ref,
             key_ref, lab_ref):
    r = roist_ref[...]
    x1, y1, x2, y2 = (r[i : i + 1, :] for i in range(4))
    g = gtall_ref[...]
    gx1, gy1, gx2, gy2 = (g[:, i : i + 1] for i in range(4))
    ix1 = jnp.maximum(x1, gx1)
    iy1 = jnp.maximum(y1, gy1)
    ix2 = jnp.minimum(x2, gx2)
    iy2 = jnp.minimum(y2, gy2)
    inter = jnp.maximum(ix2 - ix1, 0.0) * jnp.maximum(iy2 - iy1, 0.0)
    area_b = (x2 - x1) * (y2 - y1)
    area_g = (gx2 - gx1) * (gy2 - gy1)
    union = area_b + area_g - inter
    iou = inter / jnp.maximum(union, 1e-8)
    mx = jnp.max(iou, axis=0, keepdims=True)
    iota_g = lax.broadcasted_iota(jnp.int32, iou.shape, 0).astype(jnp.float32)
    am = jnp.min(jnp.where(iou == mx, iota_g, jnp.float32(G)), axis=0,
                 keepdims=True)
    onehot = iota_g == am

    def sel(c):
        col = g[:, c : c + 1]
        return jnp.sum(jnp.where(onehot, col, 0.0), axis=0, keepdims=True)

    mgx1, mgy1, mgx2, mgy2, labf = (sel(c) for c in range(5))
    ew = x2 - x1
    eh = y2 - y1
    ecx = x1 + 0.5 * ew
    ecy = y1 + 0.5 * eh
    gw = mgx2 - mgx1
    gh = mgy2 - mgy1
    gcx = mgx1 + 0.5 * gw
    gcy = mgy1 + 0.5 * gh
    ewc = jnp.maximum(ew, 1e-6)
    ehc = jnp.maximum(eh, 1e-6)
    dx_ref[...] = (gcx - ecx) / ewc / 0.1
    dy_ref[...] = (gcy - ecy) / ehc / 0.1
    dw_ref[...] = jnp.log(jnp.maximum(gw, 1e-6) / ewc) / 0.2
    dh_ref[...] = jnp.log(jnp.maximum(gh, 1e-6) / ehc) / 0.2
    key_ref[...] = ~lax.bitcast_convert_type(mx, jnp.int32)
    lab_ref[...] = labf.astype(jnp.int32)


def _tc_stage(roist, gtall):
    return pl.pallas_call(
        _tc_body,
        grid=((N + BN - 1) // BN,),
        in_specs=[
            pl.BlockSpec((4, BN), lambda i: (0, i)),
            pl.BlockSpec((G, 8), lambda i: (0, 0)),
        ],
        out_specs=[pl.BlockSpec((1, BN), lambda i: (0, i))] * 6,
        out_shape=[jax.ShapeDtypeStruct((1, N), jnp.float32)] * 4
        + [
            jax.ShapeDtypeStruct((1, N), jnp.int32),
            jax.ShapeDtypeStruct((1, N), jnp.int32),
        ],
    )(roist, gtall)


def _digit(k, p):
    kb = k - jnp.int32(_K1)
    if p == 0:
        return kb & jnp.int32(_RAD0 - 1)
    return lax.shift_right_logical(kb, jnp.int32(13))


def _scan_bins(hist, base, nbins):
    def scan_body(h, run):
        v0 = hist[pl.ds(h * 32, 16)]
        v1 = hist[pl.ds(h * 32 + 16, 16)]
        c0 = plsc.cumsum(v0)
        c1 = plsc.cumsum(v1)
        s0 = c0[15]
        base[pl.ds(h * 32, 16)] = run + c0 - v0
        base[pl.ds(h * 32 + 16, 16)] = (run + s0) + c1 - v1
        return run + s0 + c1[15]

    lax.fori_loop(0, nbins // 32, scan_body, jnp.int32(0))


def _perm_pass(src_k, src_v, dst_k, dst_v, base, p, ntv2, hist1=None):
    def perm_half(k, v):
        d = _digit(k, p)
        occ, last = plsc.scan_count(d)
        b = plsc.load_gather(base, [d])
        pos = b + occ - 1
        plsc.store_scatter(dst_k, [pos], k)
        plsc.store_scatter(dst_v, [pos], v)
        plsc.addupdate_scatter(base, [d], occ, mask=last)
        if hist1 is not None:
            d1 = _digit(k, 1)
            occ1, last1 = plsc.scan_count(d1)
            plsc.addupdate_scatter(hist1, [d1], occ1, mask=last1)

    def perm_body(t, _):
        k0 = src_k[pl.ds(t * 32, 16)]
        v0 = src_v[pl.ds(t * 32, 16)]
        k1 = src_k[pl.ds(t * 32 + 16, 16)]
        v1 = src_v[pl.ds(t * 32 + 16, 16)]
        perm_half(k0, v0)
        perm_half(k1, v1)
        return 0

    lax.fori_loop(0, ntv2, perm_body, 0)


def _sc_stage(key, zeros, dxa, dya, dwa, dha, labels_all):
    mesh = plsc.VectorSubcoreMesh(core_axis_name="c", subcore_axis_name="s")

    @functools.partial(
        pl.kernel,
        mesh=mesh,
        compiler_params=pltpu.CompilerParams(needs_layout_passes=False),
        out_type=(
            jax.ShapeDtypeStruct((N,), jnp.float32),
            jax.ShapeDtypeStruct((N,), jnp.float32),
            jax.ShapeDtypeStruct((N,), jnp.float32),
            jax.ShapeDtypeStruct((N,), jnp.float32),
            jax.ShapeDtypeStruct((N,), jnp.int32),
        ),
        scratch_types=[
            pltpu.VMEM((N + 64,), jnp.int32),
            pltpu.VMEM((N + 64,), jnp.int32),
            pltpu.VMEM((N + 64,), jnp.int32),
            pltpu.VMEM((N + 64,), jnp.int32),
            pltpu.VMEM((_RAD0,), jnp.int32),
            pltpu.VMEM((_RAD0,), jnp.int32),
            pltpu.VMEM((_RAD1,), jnp.int32),
            pltpu.VMEM((_CH,), jnp.float32),
            pltpu.VMEM((_CH,), jnp.float32),
            pltpu.VMEM((_CH,), jnp.float32),
            pltpu.VMEM((_CH,), jnp.float32),
            pltpu.VMEM((_CH,), jnp.int32),
            pltpu.VMEM((_CH,), jnp.int32),
            pltpu.SemaphoreType.DMA,
            pltpu.SemaphoreType.DMA,
            pltpu.SemaphoreType.DMA,
            pltpu.SemaphoreType.DMA,
            pltpu.SemaphoreType.DMA,
        ],
    )
    def sck(key_hbm, zeros_hbm, dx_hbm, dy_hbm, dw_hbm, dh_hbm, lall_hbm,
            dx_out, dy_out, dw_out, dh_out, lout_hbm,
            kB, vB, kC, vC, hist, base, hist1, c0, c1, c2, c3, lrows, lzero,
            sem0, sem1, sem2, sem3, sem4):
        cid = lax.axis_index("c")
        sid = lax.axis_index("s")

        @pl.when(jnp.logical_and(cid == 0, sid == 0))
        def _():
            iota16 = lax.iota(jnp.int32, 16)
            pltpu.sync_copy(zeros_hbm.at[pl.ds(0, _RAD0)], hist)
            pltpu.sync_copy(zeros_hbm.at[pl.ds(_RAD0, _RAD1)], hist1)
            pltpu.sync_copy(key_hbm, kC.at[pl.ds(0, N)])

            def compact_half(t16, off):
                k = kC[pl.ds(t16, 16)]
                sel = jnp.logical_and(
                    k < jnp.int32(_K01), k != jnp.int32(_K05)
                )
                iv = iota16 + t16
                plsc.store_compressed(kB.at[pl.ds(off, 16)], k, mask=sel)
                plsc.store_compressed(vB.at[pl.ds(off, 16)], iv, mask=sel)
                d0 = _digit(k, 0)
                occ, last = plsc.scan_count(d0, sel)
                plsc.addupdate_scatter(
                    hist, [d0], occ, mask=jnp.logical_and(last, sel)
                )
                return off + jnp.sum(sel.astype(jnp.int32))

            def compact_body(t, off):
                off = compact_half(t * 64, off)
                off = compact_half(t * 64 + 16, off)
                off = compact_half(t * 64 + 32, off)
                return compact_half(t * 64 + 48, off)

            kC[pl.ds(N, 16)] = jnp.full((16,), _K01, jnp.int32)
            kC[pl.ds(N + 16, 16)] = jnp.full((16,), _K01, jnp.int32)
            kC[pl.ds(N + 32, 16)] = jnp.full((16,), _K01, jnp.int32)
            M = lax.fori_loop(0, (N + 63) // 64, compact_body, jnp.int32(0))
            mf = (M // 16) * 16
            tailv = kB[pl.ds(mf, 16)]
            kB[pl.ds(mf, 16)] = jnp.where(
                iota16 < M - mf, tailv, jnp.int32(_K01)
            )
            kB[pl.ds(mf + 16, 16)] = jnp.full((16,), _K01, jnp.int32)
            ntv2 = (M + 31) // 32
            nsent = ntv2 * 32 - M
            hv = hist[pl.ds(_D0S_ROW, 16)]
            hist[pl.ds(_D0S_ROW, 16)] = hv + jnp.where(
                iota16 == jnp.int32(_D0S_LANE), nsent, 0
            )

            _scan_bins(hist, base, _RAD0)
            _perm_pass(kB, vB, kC, vC, base, 0, ntv2, hist1=hist1)
            _scan_bins(hist1, hist, _RAD1)
            _perm_pass(kC, vC, kB, vB, hist, 1, ntv2)
            pltpu.sync_copy(key_hbm, kC.at[pl.ds(0, N)])

            idxs = vC

            def pick_half(t16, fr, br):
                k = kB[pl.ds(t16, 16)]
                v = vB[pl.ds(t16, 16)]
                is_fg = k < jnp.int32(_K05)
                is_bg = jnp.logical_and(
                    k > jnp.int32(_K05), k < jnp.int32(_K01)
                )
                cf = plsc.cumsum(is_fg.astype(jnp.int32))
                cb = plsc.cumsum(is_bg.astype(jnp.int32))
                rf = fr + cf - 1
                rb = br + cb - 1
                plsc.store_scatter(
                    idxs, [rf], v,
                    mask=jnp.logical_and(is_fg, rf < jnp.int32(N_FG)),
                )
                plsc.store_scatter(
                    idxs, [rb + jnp.int32(N_FG)], v,
                    mask=jnp.logical_and(is_bg, rb < jnp.int32(N_BG)),
                )
                return fr + cf[15], br + cb[15]

            def pick_body(t, fb):
                fr, br = pick_half(t * 32, fb[0], fb[1])
                return pick_half(t * 32 + 16, fr, br)

            F, B = lax.fori_loop(
                0, ntv2, pick_body, (jnp.int32(0), jnp.int32(0))
            )

            def fill_half(t16, nf, nb):
                k = kC[pl.ds(t16, 16)]
                iv = iota16 + t16
                no_fg = k >= jnp.int32(_K05)
                no_bg = jnp.logical_or(
                    k <= jnp.int32(_K05), k >= jnp.int32(_K01)
                )
                cf = plsc.cumsum(no_fg.astype(jnp.int32))
                cb = plsc.cumsum(no_bg.astype(jnp.int32))
                pf = F + nf + cf - 1
                pb = B + nb + cb - 1
                plsc.store_scatter(
                    idxs, [pf], iv,
                    mask=jnp.logical_and(no_fg, pf < jnp.int32(N_FG)),
                )
                plsc.store_scatter(
                    idxs, [pb + jnp.int32(N_FG)], iv,
                    mask=jnp.logical_and(no_bg, pb < jnp.int32(N_BG)),
                )
                return nf + cf[15], nb + cb[15]

            need_f = jnp.maximum(jnp.int32(N_FG) - F, 0)
            need_b = jnp.maximum(jnp.int32(N_BG) - B, 0)

            def fill_cond(c):
                t, nf, nb = c
                return jnp.logical_and(
                    t < _NV2,
                    jnp.logical_or(nf < need_f, nb < need_b),
                )

            def fill_body(c):
                t, nf, nb = c
                nf, nb = fill_half(t * 32, nf, nb)
                nf, nb = fill_half(t * 32 + 16, nf, nb)
                return (t + 1, nf, nb)

            lax.while_loop(
                fill_cond, fill_body,
                (jnp.int32(0), jnp.int32(0), jnp.int32(0)),
            )

            fthr = jnp.minimum(F, jnp.int32(N_FG))

            def zero_body(t, _):
                lzero[pl.ds(t * 16, 16)] = jnp.zeros((16,), jnp.int32)
                return 0

            lax.fori_loop(0, _CH // 16, zero_body, 0)

            def out_body(c, _):
                gi = idxs.at[pl.ds(c * _CH, _CH)]
                cps = [
                    pltpu.make_async_copy(src.at[gi], dst, sem)
                    for src, dst, sem in (
                        (dx_hbm, c0, sem0),
                        (dy_hbm, c1, sem1),
                        (dw_hbm, c2, sem2),
                        (dh_hbm, c3, sem3),
                    )
                ]
                for cp in cps:
                    cp.start()
                need_lab = c * _CH < fthr
                sl = pl.ds(c * _CH, _CH)

                @pl.when(need_lab)
                def _():
                    cp4 = pltpu.make_async_copy(
                        lall_hbm.at[gi], lrows, sem4
                    )
                    cp4.start()
                    cp4.wait()

                    def mask_body(t, _):
                        pvec = iota16 + (c * _CH + t * 16)
                        lv = lrows[pl.ds(t * 16, 16)]
                        lrows[pl.ds(t * 16, 16)] = jnp.where(
                            pvec < fthr, lv, jnp.int32(0)
                        )
                        return 0

                    lax.fori_loop(0, _CH // 16, mask_body, 0)
                    pltpu.sync_copy(lrows, lout_hbm.at[sl])

                @pl.when(jnp.logical_not(need_lab))
                def _():
                    pltpu.sync_copy(lzero, lout_hbm.at[sl])

                for cp in cps:
                    cp.wait()
                pltpu.sync_copy(c0, dx_out.at[sl])
                pltpu.sync_copy(c1, dy_out.at[sl])
                pltpu.sync_copy(c2, dw_out.at[sl])
                pltpu.sync_copy(c3, dh_out.at[sl])
                return 0

            lax.fori_loop(0, _NCH, out_body, 0)

    return sck(key, zeros, dxa, dya, dwa, dha, labels_all)


def kernel(rois, gt_bbox, gt_labels):
    gtall = jnp.concatenate(
        [gt_bbox, gt_labels.astype(jnp.float32)[:, None],
         jnp.zeros((G, 3), jnp.float32)],
        axis=1,
    )
    dxa, dya, dwa, dha, key2d, lab2d = _tc_stage(rois.T, gtall)
    key = key2d.reshape(N)
    labels_all = lab2d.reshape(N)
    zeros = jnp.zeros((_RAD0 + _RAD1,), jnp.int32)
    dx, dy, dw, dh, labels = _sc_stage(
        key, zeros, dxa.reshape(N), dya.reshape(N), dwa.reshape(N),
        dha.reshape(N), labels_all)
    deltas = jnp.stack([dx, dy, dw, dh], axis=1)
    return deltas, labels


# speedup vs baseline: 1.0222x; 1.0113x over previous
import functools

import jax
import jax.numpy as jnp
import numpy as np
from jax import lax
from jax.experimental import pallas as pl
from jax.experimental.pallas import tpu as pltpu
from jax.experimental.pallas import tpu_sc as plsc

N = 20000
G = 128
N_FG = 5000
N_BG = 15000
BN = 2048

_K05 = int(~np.float32(0.5).view(np.int32))
_K01 = int(~np.float32(0.1).view(np.int32))
_K1 = int(~np.float32(1.0).view(np.int32))
_RAD0 = 1 << 13
_RAD1 = 1 << 12

_D0S = (_K01 - _K1) & (_RAD0 - 1)
_D0S_ROW = (_D0S // 16) * 16
_D0S_LANE = _D0S % 16

_NV = N // 16
_NV2 = _NV // 2
_CH = 4000
_NCH = N // _CH


def _tc_body(roist_ref, gtall_ref, dx_ref, dy_ref, dw_ref, dh_ref,
             key_ref, lab_ref):
    r = roist_ref[...]
    x1, y1, x2, y2 = (r[i : i + 1, :] for i in range(4))
    g = gtall_ref[...]
    gx1, gy1, gx2, gy2 = (g[:, i : i + 1] for i in range(4))
    ix1 = jnp.maximum(x1, gx1)
    iy1 = jnp.maximum(y1, gy1)
    ix2 = jnp.minimum(x2, gx2)
    iy2 = jnp.minimum(y2, gy2)
    inter = jnp.maximum(ix2 - ix1, 0.0) * jnp.maximum(iy2 - iy1, 0.0)
    area_b = (x2 - x1) * (y2 - y1)
    area_g = (gx2 - gx1) * (gy2 - gy1)
    union = area_b + area_g - inter
    iou = inter / jnp.maximum(union, 1e-8)
    mx = jnp.max(iou, axis=0, keepdims=True)
    iota_g = lax.broadcasted_iota(jnp.int32, iou.shape, 0).astype(jnp.float32)
    am = jnp.min(jnp.where(iou == mx, iota_g, jnp.float32(G)), axis=0,
                 keepdims=True)
    onehot = iota_g == am

    def sel(c):
        col = g[:, c : c + 1]
        return jnp.sum(jnp.where(onehot, col, 0.0), axis=0, keepdims=True)

    mgx1, mgy1, mgx2, mgy2, labf = (sel(c) for c in range(5))
    ew = x2 - x1
    eh = y2 - y1
    ecx = x1 + 0.5 * ew
    ecy = y1 + 0.5 * eh
    gw = mgx2 - mgx1
    gh = mgy2 - mgy1
    gcx = mgx1 + 0.5 * gw
    gcy = mgy1 + 0.5 * gh
    ewc = jnp.maximum(ew, 1e-6)
    ehc = jnp.maximum(eh, 1e-6)
    dx_ref[...] = (gcx - ecx) / ewc / 0.1
    dy_ref[...] = (gcy - ecy) / ehc / 0.1
    dw_ref[...] = jnp.log(jnp.maximum(gw, 1e-6) / ewc) / 0.2
    dh_ref[...] = jnp.log(jnp.maximum(gh, 1e-6) / ehc) / 0.2
    key_ref[...] = ~lax.bitcast_convert_type(mx, jnp.int32)
    lab_ref[...] = labf.astype(jnp.int32)


def _tc_stage(roist, gtall):
    return pl.pallas_call(
        _tc_body,
        grid=((N + BN - 1) // BN,),
        in_specs=[
            pl.BlockSpec((4, BN), lambda i: (0, i)),
            pl.BlockSpec((G, 8), lambda i: (0, 0)),
        ],
        out_specs=[pl.BlockSpec((1, BN), lambda i: (0, i))] * 6,
        out_shape=[jax.ShapeDtypeStruct((1, N), jnp.float32)] * 4
        + [
            jax.ShapeDtypeStruct((1, N), jnp.int32),
            jax.ShapeDtypeStruct((1, N), jnp.int32),
        ],
    )(roist, gtall)


def _digit(k, p):
    kb = k - jnp.int32(_K1)
    if p == 0:
        return kb & jnp.int32(_RAD0 - 1)
    return lax.shift_right_logical(kb, jnp.int32(13))


def _scan_bins(hist, base, nbins):
    def scan_body(h, run):
        v0 = hist[pl.ds(h * 32, 16)]
        v1 = hist[pl.ds(h * 32 + 16, 16)]
        c0 = plsc.cumsum(v0)
        c1 = plsc.cumsum(v1)
        s0 = c0[15]
        base[pl.ds(h * 32, 16)] = run + c0 - v0
        base[pl.ds(h * 32 + 16, 16)] = (run + s0) + c1 - v1
        return run + s0 + c1[15]

    lax.fori_loop(0, nbins // 32, scan_body, jnp.int32(0))


def _perm_pass(src_k, src_v, dst_k, dst_v, base, p, ntv2, hist1=None):
    def perm_half(k, v):
        d = _digit(k, p)
        occ, last = plsc.scan_count(d)
        b = plsc.load_gather(base, [d])
        pos = b + occ - 1
        plsc.store_scatter(dst_k, [pos], k)
        plsc.store_scatter(dst_v, [pos], v)
        plsc.addupdate_scatter(base, [d], occ, mask=last)
        if hist1 is not None:
            d1 = _digit(k, 1)
            occ1, last1 = plsc.scan_count(d1)
            plsc.addupdate_scatter(hist1, [d1], occ1, mask=last1)

    def perm_body(t, _):
        k0 = src_k[pl.ds(t * 32, 16)]
        v0 = src_v[pl.ds(t * 32, 16)]
        k1 = src_k[pl.ds(t * 32 + 16, 16)]
        v1 = src_v[pl.ds(t * 32 + 16, 16)]
        perm_half(k0, v0)
        perm_half(k1, v1)
        return 0

    lax.fori_loop(0, ntv2, perm_body, 0)


def _sc_stage(key, zeros, dxa, dya, dwa, dha, labels_all):
    mesh = plsc.VectorSubcoreMesh(core_axis_name="c", subcore_axis_name="s")

    @functools.partial(
        pl.kernel,
        mesh=mesh,
        compiler_params=pltpu.CompilerParams(needs_layout_passes=False),
        out_type=(
            jax.ShapeDtypeStruct((N,), jnp.float32),
            jax.ShapeDtypeStruct((N,), jnp.float32),
            jax.ShapeDtypeStruct((N,), jnp.float32),
            jax.ShapeDtypeStruct((N,), jnp.float32),
            jax.ShapeDtypeStruct((N,), jnp.int32),
        ),
        scratch_types=[
            pltpu.VMEM((N + 64,), jnp.int32),
            pltpu.VMEM((N + 64,), jnp.int32),
            pltpu.VMEM((N + 64,), jnp.int32),
            pltpu.VMEM((N + 64,), jnp.int32),
            pltpu.VMEM((_RAD0,), jnp.int32),
            pltpu.VMEM((_RAD0,), jnp.int32),
            pltpu.VMEM((_RAD1,), jnp.int32),
            pltpu.VMEM((_CH,), jnp.float32),
            pltpu.VMEM((_CH,), jnp.float32),
            pltpu.VMEM((_CH,), jnp.float32),
            pltpu.VMEM((_CH,), jnp.float32),
            pltpu.VMEM((_CH,), jnp.int32),
            pltpu.VMEM((_CH,), jnp.int32),
            pltpu.SemaphoreType.DMA,
            pltpu.SemaphoreType.DMA,
            pltpu.SemaphoreType.DMA,
            pltpu.SemaphoreType.DMA,
            pltpu.SemaphoreType.DMA,
        ],
    )
    def sck(key_hbm, zeros_hbm, dx_hbm, dy_hbm, dw_hbm, dh_hbm, lall_hbm,
            dx_out, dy_out, dw_out, dh_out, lout_hbm,
            kB, vB, kC, vC, hist, base, hist1, c0, c1, c2, c3, lrows, lzero,
            sem0, sem1, sem2, sem3, sem4):
        cid = lax.axis_index("c")
        sid = lax.axis_index("s")

        @pl.when(jnp.logical_and(cid == 0, sid == 0))
        def _():
            iota16 = lax.iota(jnp.int32, 16)
            pltpu.sync_copy(zeros_hbm.at[pl.ds(0, _RAD0)], hist)
            pltpu.sync_copy(zeros_hbm.at[pl.ds(_RAD0, _RAD1)], hist1)
            pltpu.sync_copy(key_hbm, kC.at[pl.ds(0, N)])

            def compact_half(t16, off):
                k = kC[pl.ds(t16, 16)]
                sel = jnp.logical_and(
                    k < jnp.int32(_K01), k != jnp.int32(_K05)
                )
                iv = iota16 + t16
                plsc.store_compressed(kB.at[pl.ds(off, 16)], k, mask=sel)
                plsc.store_compressed(vB.at[pl.ds(off, 16)], iv, mask=sel)
                d0 = _digit(k, 0)
                occ, last = plsc.scan_count(d0, sel)
                plsc.addupdate_scatter(
                    hist, [d0], occ, mask=jnp.logical_and(last, sel)
                )
                return off + jnp.sum(sel.astype(jnp.int32))

            def compact_body(t, off):
                off = compact_half(t * 64, off)
                off = compact_half(t * 64 + 16, off)
                off = compact_half(t * 64 + 32, off)
                return compact_half(t * 64 + 48, off)

            kC[pl.ds(N, 16)] = jnp.full((16,), _K01, jnp.int32)
            kC[pl.ds(N + 16, 16)] = jnp.full((16,), _K01, jnp.int32)
            kC[pl.ds(N + 32, 16)] = jnp.full((16,), _K01, jnp.int32)
            M = lax.fori_loop(0, (N + 63) // 64, compact_body, jnp.int32(0))
            mf = (M // 16) * 16
            tailv = kB[pl.ds(mf, 16)]
            kB[pl.ds(mf, 16)] = jnp.where(
                iota16 < M - mf, tailv, jnp.int32(_K01)
            )
            kB[pl.ds(mf + 16, 16)] = jnp.full((16,), _K01, jnp.int32)
            kB[pl.ds(mf + 32, 16)] = jnp.full((16,), _K01, jnp.int32)
            kB[pl.ds(mf + 48, 16)] = jnp.full((16,), _K01, jnp.int32)
            ntv4 = (M + 63) // 64
            ntv2 = ntv4 * 2
            nsent = ntv2 * 32 - M
            hv = hist[pl.ds(_D0S_ROW, 16)]
            hist[pl.ds(_D0S_ROW, 16)] = hv + jnp.where(
                iota16 == jnp.int32(_D0S_LANE), nsent, 0
            )

            _scan_bins(hist, base, _RAD0)
            _perm_pass(kB, vB, kC, vC, base, 0, ntv2, hist1=hist1)
            _scan_bins(hist1, hist, _RAD1)
            _perm_pass(kC, vC, kB, vB, hist, 1, ntv2)
            pltpu.sync_copy(key_hbm, kC.at[pl.ds(0, N)])

            idxs = vC

            def pick_half(t16, fr, br):
                k = kB[pl.ds(t16, 16)]
                v = vB[pl.ds(t16, 16)]
                is_fg = k < jnp.int32(_K05)
                is_bg = jnp.logical_and(
                    k > jnp.int32(_K05), k < jnp.int32(_K01)
                )
                cf = plsc.cumsum(is_fg.astype(jnp.int32))
                cb = plsc.cumsum(is_bg.astype(jnp.int32))
                rf = fr + cf - 1
                rb = br + cb - 1
                plsc.store_scatter(
                    idxs, [rf], v,
                    mask=jnp.logical_and(is_fg, rf < jnp.int32(N_FG)),
                )
                plsc.store_scatter(
                    idxs, [rb + jnp.int32(N_FG)], v,
                    mask=jnp.logical_and(is_bg, rb < jnp.int32(N_BG)),
                )
                return fr + cf[15], br + cb[15]

            def pick_body(t, fb):
                fr, br = pick_half(t * 64, fb[0], fb[1])
                fr, br = pick_half(t * 64 + 16, fr, br)
                fr, br = pick_half(t * 64 + 32, fr, br)
                return pick_half(t * 64 + 48, fr, br)

            F, B = lax.fori_loop(
                0, ntv4, pick_body, (jnp.int32(0), jnp.int32(0))
            )

            def fill_half(t16, nf, nb):
                k = kC[pl.ds(t16, 16)]
                iv = iota16 + t16
                no_fg = k >= jnp.int32(_K05)
                no_bg = jnp.logical_or(
                    k <= jnp.int32(_K05), k >= jnp.int32(_K01)
                )
                cf = plsc.cumsum(no_fg.astype(jnp.int32))
                cb = plsc.cumsum(no_bg.astype(jnp.int32))
                pf = F + nf + cf - 1
                pb = B + nb + cb - 1
                plsc.store_scatter(
                    idxs, [pf], iv,
                    mask=jnp.logical_and(no_fg, pf < jnp.int32(N_FG)),
                )
                plsc.store_scatter(
                    idxs, [pb + jnp.int32(N_FG)], iv,
                    mask=jnp.logical_and(no_bg, pb < jnp.int32(N_BG)),
                )
                return nf + cf[15], nb + cb[15]

            need_f = jnp.maximum(jnp.int32(N_FG) - F, 0)
            need_b = jnp.maximum(jnp.int32(N_BG) - B, 0)

            def fill_cond(c):
                t, nf, nb = c
                return jnp.logical_and(
                    t < (N + 63) // 64,
                    jnp.logical_or(nf < need_f, nb < need_b),
                )

            def fill_body(c):
                t, nf, nb = c
                nf, nb = fill_half(t * 64, nf, nb)
                nf, nb = fill_half(t * 64 + 16, nf, nb)
                nf, nb = fill_half(t * 64 + 32, nf, nb)
                nf, nb = fill_half(t * 64 + 48, nf, nb)
                return (t + 1, nf, nb)

            lax.while_loop(
                fill_cond, fill_body,
                (jnp.int32(0), jnp.int32(0), jnp.int32(0)),
            )

            fthr = jnp.minimum(F, jnp.int32(N_FG))

            def zero_body(t, _):
                lzero[pl.ds(t * 16, 16)] = jnp.zeros((16,), jnp.int32)
                return 0

            lax.fori_loop(0, _CH // 16, zero_body, 0)

            def out_body(c, _):
                gi = idxs.at[pl.ds(c * _CH, _CH)]
                cps = [
                    pltpu.make_async_copy(src.at[gi], dst, sem)
                    for src, dst, sem in (
                        (dx_hbm, c0, sem0),
                        (dy_hbm, c1, sem1),
                        (dw_hbm, c2, sem2),
                        (dh_hbm, c3, sem3),
                    )
                ]
                for cp in cps:
                    cp.start()
                need_lab = c * _CH < fthr
                sl = pl.ds(c * _CH, _CH)

                @pl.when(need_lab)
                def _():
                    cp4 = pltpu.make_async_copy(
                        lall_hbm.at[gi], lrows, sem4
                    )
                    cp4.start()
                    cp4.wait()

                    def mask_body(t, _):
                        pvec = iota16 + (c * _CH + t * 16)
                        lv = lrows[pl.ds(t * 16, 16)]
                        lrows[pl.ds(t * 16, 16)] = jnp.where(
                            pvec < fthr, lv, jnp.int32(0)
                        )
                        return 0

                    lax.fori_loop(0, _CH // 16, mask_body, 0)
                    pltpu.sync_copy(lrows, lout_hbm.at[sl])

                @pl.when(jnp.logical_not(need_lab))
                def _():
                    pltpu.sync_copy(lzero, lout_hbm.at[sl])

                for cp in cps:
                    cp.wait()
                pltpu.sync_copy(c0, dx_out.at[sl])
                pltpu.sync_copy(c1, dy_out.at[sl])
                pltpu.sync_copy(c2, dw_out.at[sl])
                pltpu.sync_copy(c3, dh_out.at[sl])
                return 0

            lax.fori_loop(0, _NCH, out_body, 0)

    return sck(key, zeros, dxa, dya, dwa, dha, labels_all)


def kernel(rois, gt_bbox, gt_labels):
    gtall = jnp.concatenate(
        [gt_bbox, gt_labels.astype(jnp.float32)[:, None],
         jnp.zeros((G, 3), jnp.float32)],
        axis=1,
    )
    dxa, dya, dwa, dha, key2d, lab2d = _tc_stage(rois.T, gtall)
    key = key2d.reshape(N)
    labels_all = lab2d.reshape(N)
    zeros = jnp.zeros((_RAD0 + _RAD1,), jnp.int32)
    dx, dy, dw, dh, labels = _sc_stage(
        key, zeros, dxa.reshape(N), dya.reshape(N), dwa.reshape(N),
        dha.reshape(N), labels_all)
    deltas = jnp.stack([dx, dy, dw, dh], axis=1)
    return deltas, labels
